# baseline (speedup 1.0000x reference)
import jax
import jax.numpy as jnp
from jax import lax
from jax.experimental import pallas as pl
from jax.experimental.pallas import tpu as pltpu
from jax.experimental.pallas import tpu_sc as plsc

_N = 10000
_E = 320000
_NC = 2
_NS = 16
_NW = _NC * _NS
_CHUNK = 128
_CPT = 80
_EPAD = _NW * _CPT * _CHUNK
_NPAD = _N + 8
_STRIPE = 624
_TAIL0 = _STRIPE * _NS
_TAIL = _N - _TAIL0
_D = 128
_PIECES = [(0, _CHUNK), (128, _CHUNK), (256, _CHUNK), (384, _CHUNK),
           (512, _STRIPE - 512)]


def _fill_rows(buf, nrows, ncols, val):
    vec = jnp.full((16,), val, jnp.float32)

    def row(r, carry):
        for j in range(ncols // 16):
            buf[r, pl.ds(j * 16, 16)] = vec
        return carry

    lax.fori_loop(0, nrows, row, 0)


def _zero_stripes(stage_v, sh, s, r0):
    _fill_rows(stage_v, _CHUNK, _D, 0.0)
    for off, n in _PIECES:
        pltpu.sync_copy(stage_v.at[pl.ds(0, n)], sh.at[pl.ds(r0 + off, n)])

    @pl.when(s == _NS - 1)
    def _():
        pltpu.sync_copy(stage_v.at[pl.ds(0, _TAIL)],
                        sh.at[pl.ds(_TAIL0, _TAIL)])


def _copy_out_stripes(stage_v, sh, out, c, s, r0):
    def piece(off, n):
        pltpu.sync_copy(sh.at[pl.ds(off, n)], stage_v.at[pl.ds(0, n)])
        pltpu.sync_copy(stage_v.at[pl.ds(0, n)], out.at[c, pl.ds(off, n)])

    for off, n in _PIECES:
        piece(r0 + off, n)

    @pl.when(s == _NS - 1)
    def _():
        piece(_TAIL0, _TAIL)


_EPT = _CPT * _CHUNK


def _seg_sum_call(table, src1, dst1):
    mesh = plsc.VectorSubcoreMesh(core_axis_name="c", subcore_axis_name="s")

    def body(table_h, src_h, dst_h, acc_out,
             sidx_v, d0_v, d1_v, rows0_v, rows1_v, gs0, gs1, ds0, ds1, acc_sh):
        c = lax.axis_index("c")
        s = lax.axis_index("s")
        wid = s * _NC + c
        r0 = s * _STRIPE
        e0 = wid * _EPT

        pltpu.sync_copy(src_h.at[pl.ds(e0, _EPT)], sidx_v)

        _zero_stripes(rows0_v, acc_sh, s, r0)
        plsc.subcore_barrier()

        def gather(t, buf, sem):
            pltpu.async_copy(table_h.at[sidx_v.at[pl.ds(t * _CHUNK, _CHUNK)]],
                             buf, sem)

        def gwait(buf, sem):
            pltpu.make_async_copy(table_h.at[pl.ds(0, _CHUNK)], buf, sem).wait()

        def dload(t, dbuf, sem):
            pltpu.async_copy(dst_h.at[pl.ds(e0 + t * _CHUNK, _CHUNK)], dbuf, sem)

        def dwait(dbuf, sem):
            pltpu.make_async_copy(dst_h.at[pl.ds(0, _CHUNK)], dbuf, sem).wait()

        def scatter(buf, dbuf):
            pltpu.sync_copy(buf, acc_sh.at[dbuf], add=True)

        dload(0, d0_v, ds0)
        dload(1, d1_v, ds1)
        gather(0, rows0_v, gs0)
        gather(1, rows1_v, gs1)

        def loop_body(k, carry):
            t = 2 * k
            gwait(rows0_v, gs0)
            dwait(d0_v, ds0)
            scatter(rows0_v, d0_v)
            gather(t + 2, rows0_v, gs0)
            dload(t + 2, d0_v, ds0)
            gwait(rows1_v, gs1)
            dwait(d1_v, ds1)
            scatter(rows1_v, d1_v)
            gather(t + 3, rows1_v, gs1)
            dload(t + 3, d1_v, ds1)
            return carry

        lax.fori_loop(0, _CPT // 2 - 1, loop_body, 0)

        gwait(rows0_v, gs0)
        dwait(d0_v, ds0)
        scatter(rows0_v, d0_v)
        gwait(rows1_v, gs1)
        dwait(d1_v, ds1)
        scatter(rows1_v, d1_v)

        plsc.subcore_barrier()
        _copy_out_stripes(rows0_v, acc_sh, acc_out, c, s, r0)

    fn = pl.kernel(
        body,
        out_type=(jax.ShapeDtypeStruct((_NC, _N, _D), jnp.float32),),
        mesh=mesh,
        scratch_types=(
            pltpu.VMEM((_EPT,), jnp.int32),
            pltpu.VMEM((_CHUNK,), jnp.int32),
            pltpu.VMEM((_CHUNK,), jnp.int32),
            pltpu.VMEM((_CHUNK, _D), jnp.float32),
            pltpu.VMEM((_CHUNK, _D), jnp.float32),
            pltpu.SemaphoreType.DMA,
            pltpu.SemaphoreType.DMA,
            pltpu.SemaphoreType.DMA,
            pltpu.SemaphoreType.DMA,
            pltpu.VMEM_SHARED((_NPAD, _D), jnp.float32),
        ),
    )
    return fn(table, src1, dst1)[0]


def _deg_call(dst1):
    mesh = plsc.VectorSubcoreMesh(core_axis_name="c", subcore_axis_name="s")

    def body(dst_h, deg_out, d0_v, d1_v, ones_v, ds0, ds1, deg_sh):
        c = lax.axis_index("c")
        s = lax.axis_index("s")
        wid = s * _NC + c
        r0 = s * _STRIPE
        e0 = wid * _EPT

        _zero_stripes(ones_v, deg_sh, s, r0)
        _fill_rows(ones_v, _CHUNK, _D, 1.0)
        plsc.subcore_barrier()

        def dload(t, dbuf, sem):
            pltpu.async_copy(dst_h.at[pl.ds(e0 + t * _CHUNK, _CHUNK)], dbuf, sem)

        def dwait(dbuf, sem):
            pltpu.make_async_copy(dst_h.at[pl.ds(0, _CHUNK)], dbuf, sem).wait()

        dload(0, d0_v, ds0)
        dload(1, d1_v, ds1)

        def loop_body(k, carry):
            t = 2 * k
            dwait(d0_v, ds0)
            pltpu.sync_copy(ones_v, deg_sh.at[d0_v], add=True)
            dload(t + 2, d0_v, ds0)
            dwait(d1_v, ds1)
            pltpu.sync_copy(ones_v, deg_sh.at[d1_v], add=True)
            dload(t + 3, d1_v, ds1)
            return carry

        lax.fori_loop(0, _CPT // 2 - 1, loop_body, 0)

        dwait(d0_v, ds0)
        pltpu.sync_copy(ones_v, deg_sh.at[d0_v], add=True)
        dwait(d1_v, ds1)
        pltpu.sync_copy(ones_v, deg_sh.at[d1_v], add=True)

        plsc.subcore_barrier()
        _copy_out_stripes(ones_v, deg_sh, deg_out, c, s, r0)

    fn = pl.kernel(
        body,
        out_type=(jax.ShapeDtypeStruct((_NC, _N, _D), jnp.float32),),
        mesh=mesh,
        scratch_types=(
            pltpu.VMEM((_CHUNK,), jnp.int32),
            pltpu.VMEM((_CHUNK,), jnp.int32),
            pltpu.VMEM((_CHUNK, _D), jnp.float32),
            pltpu.SemaphoreType.DMA,
            pltpu.SemaphoreType.DMA,
            pltpu.VMEM_SHARED((_NPAD, _D), jnp.float32),
        ),
    )
    return fn(dst1)[0]


_R = 1000
_IW = 8


def _tc1_body(agg_ref, deg_ref, x_ref, wl1_ref, bl1_ref, wr1_ref,
              wr2_ref, h_ref, hr_ref, invd_ref):
    a = agg_ref[0] + agg_ref[1]
    d = deg_ref[0, :, 0:1] + deg_ref[1, :, 0:1]
    invd = 1.0 / jnp.maximum(d, 1.0)
    mean = a * invd
    h = mean @ wl1_ref[...] + bl1_ref[...] + x_ref[...] @ wr1_ref[...]
    h = jnp.maximum(h, 0.0)
    h_ref[...] = h
    hr_ref[...] = h @ wr2_ref[...]
    invd_ref[...] = jnp.broadcast_to(invd, (invd.shape[0], _IW))


def _tc1(aggp, degp, x, Wl1T, bl1, Wr1T, Wr2T):
    grid = (_N // _R,)
    D_IN, D_OUT = 128, 64
    return pl.pallas_call(
        _tc1_body,
        grid=grid,
        in_specs=[
            pl.BlockSpec((_NC, _R, D_IN), lambda i: (0, i, 0)),
            pl.BlockSpec((_NC, _R, _D), lambda i: (0, i, 0)),
            pl.BlockSpec((_R, D_IN), lambda i: (i, 0)),
            pl.BlockSpec((D_IN, D_IN), lambda i: (0, 0)),
            pl.BlockSpec((1, D_IN), lambda i: (0, 0)),
            pl.BlockSpec((D_IN, D_IN), lambda i: (0, 0)),
            pl.BlockSpec((D_IN, D_OUT), lambda i: (0, 0)),
        ],
        out_specs=[
            pl.BlockSpec((_R, D_IN), lambda i: (i, 0)),
            pl.BlockSpec((_R, D_OUT), lambda i: (i, 0)),
            pl.BlockSpec((_R, _IW), lambda i: (i, 0)),
        ],
        out_shape=[
            jax.ShapeDtypeStruct((_N, D_IN), jnp.float32),
            jax.ShapeDtypeStruct((_N, D_OUT), jnp.float32),
            jax.ShapeDtypeStruct((_N, _IW), jnp.float32),
        ],
    )(aggp, degp, x, Wl1T, bl1, Wr1T, Wr2T)


def _tc2_body(agg_ref, invd_ref, hr_ref, wl2_ref, bl2_ref, wc1_ref, bc1_ref,
              wc2_ref, bc2_ref, emb_ref, log_ref, prob_ref):
    a = agg_ref[0] + agg_ref[1]
    mean = a * invd_ref[:, 0:1]
    emb = mean @ wl2_ref[...] + bl2_ref[...] + hr_ref[...]
    z = jnp.maximum(emb @ wc1_ref[...] + bc1_ref[...], 0.0)
    logits = z @ wc2_ref[...] + bc2_ref[...]
    m = jnp.max(logits, axis=1, keepdims=True)
    e = jnp.exp(logits - m)
    probs = e / jnp.sum(e, axis=1, keepdims=True)
    emb_ref[...] = emb
    log_ref[...] = logits
    prob_ref[...] = probs


def _tc2(agg2p, invd, hr, Wl2T, bl2, Wc1T, bc1, Wc2T, bc2):
    grid = (_N // _R,)
    D_IN, D_OUT, D_C = 128, 64, 64
    return pl.pallas_call(
        _tc2_body,
        grid=grid,
        in_specs=[
            pl.BlockSpec((_NC, _R, D_IN), lambda i: (0, i, 0)),
            pl.BlockSpec((_R, _IW), lambda i: (i, 0)),
            pl.BlockSpec((_R, D_OUT), lambda i: (i, 0)),
            pl.BlockSpec((D_IN, D_OUT), lambda i: (0, 0)),
            pl.BlockSpec((1, D_OUT), lambda i: (0, 0)),
            pl.BlockSpec((D_OUT, D_C), lambda i: (0, 0)),
            pl.BlockSpec((1, D_C), lambda i: (0, 0)),
            pl.BlockSpec((D_C, 2), lambda i: (0, 0)),
            pl.BlockSpec((1, 2), lambda i: (0, 0)),
        ],
        out_specs=[
            pl.BlockSpec((_R, D_OUT), lambda i: (i, 0)),
            pl.BlockSpec((_R, 2), lambda i: (i, 0)),
            pl.BlockSpec((_R, 2), lambda i: (i, 0)),
        ],
        out_shape=[
            jax.ShapeDtypeStruct((_N, D_OUT), jnp.float32),
            jax.ShapeDtypeStruct((_N, 2), jnp.float32),
            jax.ShapeDtypeStruct((_N, 2), jnp.float32),
        ],
    )(agg2p, invd, hr, Wl2T, bl2, Wc1T, bc1, Wc2T, bc2)


@jax.jit
def kernel(x, edge_index, W_l1, b_l1, W_r1, W_l2, b_l2, W_r2,
           Wc1, bc1, Wc2, bc2):
    npad = _EPAD - _E
    src1 = jnp.concatenate([edge_index[0], jnp.zeros((npad,), jnp.int32)])
    dst1 = jnp.concatenate([edge_index[1], jnp.full((npad,), _N, jnp.int32)])

    agg1p = _seg_sum_call(x, src1, dst1)
    degp = _deg_call(dst1)
    h, hr, invd = _tc1(agg1p, degp, x, W_l1.T, b_l1[None, :], W_r1.T, W_r2.T)
    agg2p = _seg_sum_call(h, src1, dst1)
    emb, logits, probs = _tc2(agg2p, invd, hr, W_l2.T, b_l2[None, :],
                              Wc1.T, bc1[None, :], Wc2.T, bc2[None, :])
    return logits, emb, probs


# speedup vs baseline: 3.2568x; 3.2568x over previous
import jax
import jax.numpy as jnp
from jax import lax
from jax.experimental import pallas as pl
from jax.experimental.pallas import tpu as pltpu
from jax.experimental.pallas import tpu_sc as plsc

_N = 10000
_E = 320000
_NC = 2
_NS = 16
_NW = _NC * _NS
_CHUNK = 128
_CPT = 80
_EPAD = _NW * _CPT * _CHUNK
_NPAD = _N + 128
_STRIPE = 624
_TAIL0 = _STRIPE * _NS
_TAIL = _N - _TAIL0
_D = 128
_PIECES = [(0, _CHUNK), (128, _CHUNK), (256, _CHUNK), (384, _CHUNK),
           (512, _STRIPE - 512)]


def _fill_rows(buf, nrows, ncols, val):
    vec = jnp.full((16,), val, jnp.float32)

    def row(r, carry):
        for j in range(ncols // 16):
            buf[r, pl.ds(j * 16, 16)] = vec
        return carry

    lax.fori_loop(0, nrows, row, 0)


def _zero_stripes(stage_v, sh, s, r0):
    _fill_rows(stage_v, _CHUNK, _D, 0.0)
    for off, n in _PIECES:
        pltpu.sync_copy(stage_v.at[pl.ds(0, n)], sh.at[pl.ds(r0 + off, n)])

    @pl.when(s == _NS - 1)
    def _():
        pltpu.sync_copy(stage_v.at[pl.ds(0, _TAIL)],
                        sh.at[pl.ds(_TAIL0, _TAIL)])


def _copy_out_stripes(stage_v, sh, out, c, s, r0):
    def piece(off, n):
        pltpu.sync_copy(sh.at[pl.ds(off, n)], stage_v.at[pl.ds(0, n)])
        pltpu.sync_copy(stage_v.at[pl.ds(0, n)], out.at[c, pl.ds(off, n)])

    for off, n in _PIECES:
        piece(r0 + off, n)

    @pl.when(s == _NS - 1)
    def _():
        piece(_TAIL0, _TAIL)


_EPT = _CPT * _CHUNK


def _seg_sum_call(table, src1, dst1):
    mesh = plsc.VectorSubcoreMesh(core_axis_name="c", subcore_axis_name="s")

    def body(table_h, src_h, dst_h, acc_out,
             sidx_v, d0_v, d1_v, rows0_v, rows1_v, gs0, gs1, ds0, ds1, acc_sh):
        c = lax.axis_index("c")
        s = lax.axis_index("s")
        wid = s * _NC + c
        r0 = s * _STRIPE
        e0 = wid * _EPT

        pltpu.sync_copy(src_h.at[pl.ds(e0, _EPT)], sidx_v)

        _zero_stripes(rows0_v, acc_sh, s, r0)
        plsc.subcore_barrier()

        def gather(t, buf, sem):
            pltpu.async_copy(table_h.at[sidx_v.at[pl.ds(t * _CHUNK, _CHUNK)]],
                             buf, sem)

        def gwait(buf, sem):
            pltpu.make_async_copy(table_h.at[pl.ds(0, _CHUNK)], buf, sem).wait()

        def dload(t, dbuf, sem):
            pltpu.async_copy(dst_h.at[pl.ds(e0 + t * _CHUNK, _CHUNK)], dbuf, sem)

        def dwait(dbuf, sem):
            pltpu.make_async_copy(dst_h.at[pl.ds(0, _CHUNK)], dbuf, sem).wait()

        def scatter(buf, dbuf):
            pltpu.sync_copy(buf, acc_sh.at[dbuf], add=True)

        dload(0, d0_v, ds0)
        dload(1, d1_v, ds1)
        gather(0, rows0_v, gs0)
        gather(1, rows1_v, gs1)

        def loop_body(k, carry):
            t = 2 * k
            gwait(rows0_v, gs0)
            dwait(d0_v, ds0)
            scatter(rows0_v, d0_v)
            gather(t + 2, rows0_v, gs0)
            dload(t + 2, d0_v, ds0)
            gwait(rows1_v, gs1)
            dwait(d1_v, ds1)
            scatter(rows1_v, d1_v)
            gather(t + 3, rows1_v, gs1)
            dload(t + 3, d1_v, ds1)
            return carry

        lax.fori_loop(0, _CPT // 2 - 1, loop_body, 0)

        gwait(rows0_v, gs0)
        dwait(d0_v, ds0)
        scatter(rows0_v, d0_v)
        gwait(rows1_v, gs1)
        dwait(d1_v, ds1)
        scatter(rows1_v, d1_v)

        plsc.subcore_barrier()
        _copy_out_stripes(rows0_v, acc_sh, acc_out, c, s, r0)

    fn = pl.kernel(
        body,
        out_type=(jax.ShapeDtypeStruct((_NC, _N, _D), jnp.float32),),
        mesh=mesh,
        scratch_types=(
            pltpu.VMEM((_EPT,), jnp.int32),
            pltpu.VMEM((_CHUNK,), jnp.int32),
            pltpu.VMEM((_CHUNK,), jnp.int32),
            pltpu.VMEM((_CHUNK, _D), jnp.float32),
            pltpu.VMEM((_CHUNK, _D), jnp.float32),
            pltpu.SemaphoreType.DMA,
            pltpu.SemaphoreType.DMA,
            pltpu.SemaphoreType.DMA,
            pltpu.SemaphoreType.DMA,
            pltpu.VMEM_SHARED((_NPAD, _D), jnp.float32),
        ),
    )
    return fn(table, src1, dst1)[0]


def _deg_call(dst1):
    mesh = plsc.VectorSubcoreMesh(core_axis_name="c", subcore_axis_name="s")

    def body(dst_h, deg_out, d0_v, d1_v, ones_v, ds0, ds1, deg_sh):
        c = lax.axis_index("c")
        s = lax.axis_index("s")
        wid = s * _NC + c
        r0 = s * _STRIPE
        e0 = wid * _EPT

        _zero_stripes(ones_v, deg_sh, s, r0)
        _fill_rows(ones_v, _CHUNK, _D, 1.0)
        plsc.subcore_barrier()

        def dload(t, dbuf, sem):
            pltpu.async_copy(dst_h.at[pl.ds(e0 + t * _CHUNK, _CHUNK)], dbuf, sem)

        def dwait(dbuf, sem):
            pltpu.make_async_copy(dst_h.at[pl.ds(0, _CHUNK)], dbuf, sem).wait()

        dload(0, d0_v, ds0)
        dload(1, d1_v, ds1)

        def loop_body(k, carry):
            t = 2 * k
            dwait(d0_v, ds0)
            pltpu.sync_copy(ones_v, deg_sh.at[d0_v], add=True)
            dload(t + 2, d0_v, ds0)
            dwait(d1_v, ds1)
            pltpu.sync_copy(ones_v, deg_sh.at[d1_v], add=True)
            dload(t + 3, d1_v, ds1)
            return carry

        lax.fori_loop(0, _CPT // 2 - 1, loop_body, 0)

        dwait(d0_v, ds0)
        pltpu.sync_copy(ones_v, deg_sh.at[d0_v], add=True)
        dwait(d1_v, ds1)
        pltpu.sync_copy(ones_v, deg_sh.at[d1_v], add=True)

        plsc.subcore_barrier()
        _copy_out_stripes(ones_v, deg_sh, deg_out, c, s, r0)

    fn = pl.kernel(
        body,
        out_type=(jax.ShapeDtypeStruct((_NC, _N, _D), jnp.float32),),
        mesh=mesh,
        scratch_types=(
            pltpu.VMEM((_CHUNK,), jnp.int32),
            pltpu.VMEM((_CHUNK,), jnp.int32),
            pltpu.VMEM((_CHUNK, _D), jnp.float32),
            pltpu.SemaphoreType.DMA,
            pltpu.SemaphoreType.DMA,
            pltpu.VMEM_SHARED((_NPAD, _D), jnp.float32),
        ),
    )
    return fn(dst1)[0]


_R = 1000
_IW = 8


def _tc1_body(agg_ref, deg_ref, x_ref, wl1_ref, bl1_ref, wr1_ref,
              wr2_ref, h_ref, hr_ref, invd_ref):
    a = agg_ref[0] + agg_ref[1]
    d = deg_ref[0, :, 0:1] + deg_ref[1, :, 0:1]
    invd = 1.0 / jnp.maximum(d, 1.0)
    mean = a * invd
    h = mean @ wl1_ref[...] + bl1_ref[...] + x_ref[...] @ wr1_ref[...]
    h = jnp.maximum(h, 0.0)
    h_ref[...] = h
    hr_ref[...] = h @ wr2_ref[...]
    invd_ref[...] = jnp.broadcast_to(invd, (invd.shape[0], _IW))


def _tc1(aggp, degp, x, Wl1T, bl1, Wr1T, Wr2T):
    grid = (_N // _R,)
    D_IN, D_OUT = 128, 64
    return pl.pallas_call(
        _tc1_body,
        grid=grid,
        in_specs=[
            pl.BlockSpec((_NC, _R, D_IN), lambda i: (0, i, 0)),
            pl.BlockSpec((_NC, _R, _D), lambda i: (0, i, 0)),
            pl.BlockSpec((_R, D_IN), lambda i: (i, 0)),
            pl.BlockSpec((D_IN, D_IN), lambda i: (0, 0)),
            pl.BlockSpec((1, D_IN), lambda i: (0, 0)),
            pl.BlockSpec((D_IN, D_IN), lambda i: (0, 0)),
            pl.BlockSpec((D_IN, D_OUT), lambda i: (0, 0)),
        ],
        out_specs=[
            pl.BlockSpec((_R, D_IN), lambda i: (i, 0)),
            pl.BlockSpec((_R, D_OUT), lambda i: (i, 0)),
            pl.BlockSpec((_R, _IW), lambda i: (i, 0)),
        ],
        out_shape=[
            jax.ShapeDtypeStruct((_N, D_IN), jnp.float32),
            jax.ShapeDtypeStruct((_N, D_OUT), jnp.float32),
            jax.ShapeDtypeStruct((_N, _IW), jnp.float32),
        ],
    )(aggp, degp, x, Wl1T, bl1, Wr1T, Wr2T)


def _tc2_body(agg_ref, invd_ref, hr_ref, wl2_ref, bl2_ref, wc1_ref, bc1_ref,
              wc2_ref, bc2_ref, emb_ref, log_ref, prob_ref):
    a = agg_ref[0] + agg_ref[1]
    mean = a * invd_ref[:, 0:1]
    emb = mean @ wl2_ref[...] + bl2_ref[...] + hr_ref[...]
    z = jnp.maximum(emb @ wc1_ref[...] + bc1_ref[...], 0.0)
    logits = z @ wc2_ref[...] + bc2_ref[...]
    m = jnp.max(logits, axis=1, keepdims=True)
    e = jnp.exp(logits - m)
    probs = e / jnp.sum(e, axis=1, keepdims=True)
    emb_ref[...] = emb
    log_ref[...] = logits
    prob_ref[...] = probs


def _tc2(agg2p, invd, hr, Wl2T, bl2, Wc1T, bc1, Wc2T, bc2):
    grid = (_N // _R,)
    D_IN, D_OUT, D_C = 128, 64, 64
    return pl.pallas_call(
        _tc2_body,
        grid=grid,
        in_specs=[
            pl.BlockSpec((_NC, _R, D_IN), lambda i: (0, i, 0)),
            pl.BlockSpec((_R, _IW), lambda i: (i, 0)),
            pl.BlockSpec((_R, D_OUT), lambda i: (i, 0)),
            pl.BlockSpec((D_IN, D_OUT), lambda i: (0, 0)),
            pl.BlockSpec((1, D_OUT), lambda i: (0, 0)),
            pl.BlockSpec((D_OUT, D_C), lambda i: (0, 0)),
            pl.BlockSpec((1, D_C), lambda i: (0, 0)),
            pl.BlockSpec((D_C, 2), lambda i: (0, 0)),
            pl.BlockSpec((1, 2), lambda i: (0, 0)),
        ],
        out_specs=[
            pl.BlockSpec((_R, D_OUT), lambda i: (i, 0)),
            pl.BlockSpec((_R, 2), lambda i: (i, 0)),
            pl.BlockSpec((_R, 2), lambda i: (i, 0)),
        ],
        out_shape=[
            jax.ShapeDtypeStruct((_N, D_OUT), jnp.float32),
            jax.ShapeDtypeStruct((_N, 2), jnp.float32),
            jax.ShapeDtypeStruct((_N, 2), jnp.float32),
        ],
    )(agg2p, invd, hr, Wl2T, bl2, Wc1T, bc1, Wc2T, bc2)


@jax.jit
def kernel(x, edge_index, W_l1, b_l1, W_r1, W_l2, b_l2, W_r2,
           Wc1, bc1, Wc2, bc2):
    npad = _EPAD - _E
    pad_i = jnp.arange(npad, dtype=jnp.int32)
    src1 = jnp.concatenate([edge_index[0], (pad_i * 127) % _N])
    dst1 = jnp.concatenate([edge_index[1], _N + pad_i % 128])

    agg1p = _seg_sum_call(x, src1, dst1)
    degp = _deg_call(dst1)
    h, hr, invd = _tc1(agg1p, degp, x, W_l1.T, b_l1[None, :], W_r1.T, W_r2.T)
    agg2p = _seg_sum_call(h, src1, dst1)
    emb, logits, probs = _tc2(agg2p, invd, hr, W_l2.T, b_l2[None, :],
                              Wc1.T, bc1[None, :], Wc2.T, bc2[None, :])
    return logits, emb, probs


# speedup vs baseline: 3.2895x; 1.0100x over previous
import jax
import jax.numpy as jnp
from jax import lax
from jax.experimental import pallas as pl
from jax.experimental.pallas import tpu as pltpu
from jax.experimental.pallas import tpu_sc as plsc

_N = 10000
_E = 320000
_NC = 2
_NS = 16
_NW = _NC * _NS
_CHUNK = 128
_CPT = 80
_EPAD = _NW * _CPT * _CHUNK
_NPAD = _N + 128
_STRIPE = 624
_TAIL0 = _STRIPE * _NS
_TAIL = _N - _TAIL0
_D = 128
_PIECES = [(0, _CHUNK), (128, _CHUNK), (256, _CHUNK), (384, _CHUNK),
           (512, _STRIPE - 512)]


def _fill_rows(buf, nrows, ncols, val):
    vec = jnp.full((16,), val, jnp.float32)

    def row(r, carry):
        for j in range(ncols // 16):
            buf[r, pl.ds(j * 16, 16)] = vec
        return carry

    lax.fori_loop(0, nrows, row, 0)


def _zero_stripes(stage_v, sh, s, r0):
    _fill_rows(stage_v, _CHUNK, _D, 0.0)
    for off, n in _PIECES:
        pltpu.sync_copy(stage_v.at[pl.ds(0, n)], sh.at[pl.ds(r0 + off, n)])

    @pl.when(s == _NS - 1)
    def _():
        pltpu.sync_copy(stage_v.at[pl.ds(0, _TAIL)],
                        sh.at[pl.ds(_TAIL0, _TAIL)])


def _copy_out_stripes(stage_v, sh, out, c, s, r0):
    def piece(off, n):
        pltpu.sync_copy(sh.at[pl.ds(off, n)], stage_v.at[pl.ds(0, n)])
        pltpu.sync_copy(stage_v.at[pl.ds(0, n)], out.at[c, pl.ds(off, n)])

    for off, n in _PIECES:
        piece(r0 + off, n)

    @pl.when(s == _NS - 1)
    def _():
        piece(_TAIL0, _TAIL)


_EPT = _CPT * _CHUNK


def _seg_sum_call(table, src1, dst1, with_deg=False):
    mesh = plsc.VectorSubcoreMesh(core_axis_name="c", subcore_axis_name="s")

    def body(table_h, src_h, dst_h, acc_out, deg_out,
             sidx_v, d0_v, d1_v, rows0_v, rows1_v, gs0, gs1, ds0, ds1, acc_sh):
        c = lax.axis_index("c")
        s = lax.axis_index("s")
        wid = s * _NC + c
        r0 = s * _STRIPE
        e0 = wid * _EPT
        with_deg = deg_out is not None

        pltpu.sync_copy(src_h.at[pl.ds(e0, _EPT)], sidx_v)

        _zero_stripes(rows0_v, acc_sh, s, r0)
        plsc.subcore_barrier()

        def gather(t, buf, sem):
            pltpu.async_copy(table_h.at[sidx_v.at[pl.ds(t * _CHUNK, _CHUNK)]],
                             buf, sem)

        def gwait(buf, sem):
            pltpu.make_async_copy(table_h.at[pl.ds(0, _CHUNK)], buf, sem).wait()

        def dload(t, dbuf, sem):
            pltpu.async_copy(dst_h.at[pl.ds(e0 + t * _CHUNK, _CHUNK)], dbuf, sem)

        def dwait(dbuf, sem):
            pltpu.make_async_copy(dst_h.at[pl.ds(0, _CHUNK)], dbuf, sem).wait()

        def scatter(buf, dbuf):
            pltpu.sync_copy(buf, acc_sh.at[dbuf], add=True)

        dload(0, d0_v, ds0)
        dload(1, d1_v, ds1)
        gather(0, rows0_v, gs0)
        gather(1, rows1_v, gs1)

        def loop_body(k, carry):
            t = 2 * k
            gwait(rows0_v, gs0)
            dwait(d0_v, ds0)
            scatter(rows0_v, d0_v)
            gather(t + 2, rows0_v, gs0)
            dload(t + 2, d0_v, ds0)
            gwait(rows1_v, gs1)
            dwait(d1_v, ds1)
            scatter(rows1_v, d1_v)
            gather(t + 3, rows1_v, gs1)
            dload(t + 3, d1_v, ds1)
            return carry

        lax.fori_loop(0, _CPT // 2 - 1, loop_body, 0)

        gwait(rows0_v, gs0)
        dwait(d0_v, ds0)
        scatter(rows0_v, d0_v)
        gwait(rows1_v, gs1)
        dwait(d1_v, ds1)
        scatter(rows1_v, d1_v)

        plsc.subcore_barrier()
        _copy_out_stripes(rows0_v, acc_sh, acc_out, c, s, r0)

        if with_deg:
            plsc.subcore_barrier()
            _zero_stripes(rows0_v, acc_sh, s, r0)
            _fill_rows(rows1_v, _CHUNK, _D, 1.0)
            plsc.subcore_barrier()

            dload(0, d0_v, ds0)
            dload(1, d1_v, ds1)

            def deg_body(k, carry):
                t = 2 * k
                dwait(d0_v, ds0)
                scatter(rows1_v, d0_v)
                dload(t + 2, d0_v, ds0)
                dwait(d1_v, ds1)
                scatter(rows1_v, d1_v)
                dload(t + 3, d1_v, ds1)
                return carry

            lax.fori_loop(0, _CPT // 2 - 1, deg_body, 0)

            dwait(d0_v, ds0)
            scatter(rows1_v, d0_v)
            dwait(d1_v, ds1)
            scatter(rows1_v, d1_v)

            plsc.subcore_barrier()
            _copy_out_stripes(rows0_v, acc_sh, deg_out, c, s, r0)

    def body_with_deg(*refs):
        body(*refs[:5], *refs[5:])

    def body_no_deg(table_h, src_h, dst_h, acc_out, *rest):
        body(table_h, src_h, dst_h, acc_out, None, *rest)

    out_type = [jax.ShapeDtypeStruct((_NC, _N, _D), jnp.float32)]
    if with_deg:
        out_type.append(jax.ShapeDtypeStruct((_NC, _N, _D), jnp.float32))
    fn = pl.kernel(
        body_with_deg if with_deg else body_no_deg,
        out_type=tuple(out_type),
        mesh=mesh,
        scratch_types=(
            pltpu.VMEM((_EPT,), jnp.int32),
            pltpu.VMEM((_CHUNK,), jnp.int32),
            pltpu.VMEM((_CHUNK,), jnp.int32),
            pltpu.VMEM((_CHUNK, _D), jnp.float32),
            pltpu.VMEM((_CHUNK, _D), jnp.float32),
            pltpu.SemaphoreType.DMA,
            pltpu.SemaphoreType.DMA,
            pltpu.SemaphoreType.DMA,
            pltpu.SemaphoreType.DMA,
            pltpu.VMEM_SHARED((_NPAD, _D), jnp.float32),
        ),
    )
    out = fn(table, src1, dst1)
    return out if with_deg else out[0]


_R = 1000
_IW = 8


def _tc1_body(agg_ref, deg_ref, x_ref, wl1_ref, bl1_ref, wr1_ref,
              wr2_ref, h_ref, hr_ref, invd_ref):
    a = agg_ref[0] + agg_ref[1]
    d = deg_ref[0, :, 0:1] + deg_ref[1, :, 0:1]
    invd = 1.0 / jnp.maximum(d, 1.0)
    mean = a * invd
    h = mean @ wl1_ref[...] + bl1_ref[...] + x_ref[...] @ wr1_ref[...]
    h = jnp.maximum(h, 0.0)
    h_ref[...] = h
    hr_ref[...] = h @ wr2_ref[...]
    invd_ref[...] = jnp.broadcast_to(invd, (invd.shape[0], _IW))


def _tc1(aggp, degp, x, Wl1T, bl1, Wr1T, Wr2T):
    grid = (_N // _R,)
    D_IN, D_OUT = 128, 64
    return pl.pallas_call(
        _tc1_body,
        grid=grid,
        in_specs=[
            pl.BlockSpec((_NC, _R, D_IN), lambda i: (0, i, 0)),
            pl.BlockSpec((_NC, _R, _D), lambda i: (0, i, 0)),
            pl.BlockSpec((_R, D_IN), lambda i: (i, 0)),
            pl.BlockSpec((D_IN, D_IN), lambda i: (0, 0)),
            pl.BlockSpec((1, D_IN), lambda i: (0, 0)),
            pl.BlockSpec((D_IN, D_IN), lambda i: (0, 0)),
            pl.BlockSpec((D_IN, D_OUT), lambda i: (0, 0)),
        ],
        out_specs=[
            pl.BlockSpec((_R, D_IN), lambda i: (i, 0)),
            pl.BlockSpec((_R, D_OUT), lambda i: (i, 0)),
            pl.BlockSpec((_R, _IW), lambda i: (i, 0)),
        ],
        out_shape=[
            jax.ShapeDtypeStruct((_N, D_IN), jnp.float32),
            jax.ShapeDtypeStruct((_N, D_OUT), jnp.float32),
            jax.ShapeDtypeStruct((_N, _IW), jnp.float32),
        ],
    )(aggp, degp, x, Wl1T, bl1, Wr1T, Wr2T)


def _tc2_body(agg_ref, invd_ref, hr_ref, wl2_ref, bl2_ref, wc1_ref, bc1_ref,
              wc2_ref, bc2_ref, emb_ref, log_ref, prob_ref):
    a = agg_ref[0] + agg_ref[1]
    mean = a * invd_ref[:, 0:1]
    emb = mean @ wl2_ref[...] + bl2_ref[...] + hr_ref[...]
    z = jnp.maximum(emb @ wc1_ref[...] + bc1_ref[...], 0.0)
    logits = z @ wc2_ref[...] + bc2_ref[...]
    m = jnp.max(logits, axis=1, keepdims=True)
    e = jnp.exp(logits - m)
    probs = e / jnp.sum(e, axis=1, keepdims=True)
    emb_ref[...] = emb
    log_ref[...] = logits
    prob_ref[...] = probs


def _tc2(agg2p, invd, hr, Wl2T, bl2, Wc1T, bc1, Wc2T, bc2):
    grid = (_N // _R,)
    D_IN, D_OUT, D_C = 128, 64, 64
    return pl.pallas_call(
        _tc2_body,
        grid=grid,
        in_specs=[
            pl.BlockSpec((_NC, _R, D_IN), lambda i: (0, i, 0)),
            pl.BlockSpec((_R, _IW), lambda i: (i, 0)),
            pl.BlockSpec((_R, D_OUT), lambda i: (i, 0)),
            pl.BlockSpec((D_IN, D_OUT), lambda i: (0, 0)),
            pl.BlockSpec((1, D_OUT), lambda i: (0, 0)),
            pl.BlockSpec((D_OUT, D_C), lambda i: (0, 0)),
            pl.BlockSpec((1, D_C), lambda i: (0, 0)),
            pl.BlockSpec((D_C, 2), lambda i: (0, 0)),
            pl.BlockSpec((1, 2), lambda i: (0, 0)),
        ],
        out_specs=[
            pl.BlockSpec((_R, D_OUT), lambda i: (i, 0)),
            pl.BlockSpec((_R, 2), lambda i: (i, 0)),
            pl.BlockSpec((_R, 2), lambda i: (i, 0)),
        ],
        out_shape=[
            jax.ShapeDtypeStruct((_N, D_OUT), jnp.float32),
            jax.ShapeDtypeStruct((_N, 2), jnp.float32),
            jax.ShapeDtypeStruct((_N, 2), jnp.float32),
        ],
    )(agg2p, invd, hr, Wl2T, bl2, Wc1T, bc1, Wc2T, bc2)


@jax.jit
def kernel(x, edge_index, W_l1, b_l1, W_r1, W_l2, b_l2, W_r2,
           Wc1, bc1, Wc2, bc2):
    npad = _EPAD - _E
    pad_i = jnp.arange(npad, dtype=jnp.int32)
    src1 = jnp.concatenate([edge_index[0], (pad_i * 127) % _N])
    dst1 = jnp.concatenate([edge_index[1], _N + pad_i % 128])

    agg1p, degp = _seg_sum_call(x, src1, dst1, with_deg=True)
    h, hr, invd = _tc1(agg1p, degp, x, W_l1.T, b_l1[None, :], W_r1.T, W_r2.T)
    agg2p = _seg_sum_call(h, src1, dst1)
    emb, logits, probs = _tc2(agg2p, invd, hr, W_l2.T, b_l2[None, :],
                              Wc1.T, bc1[None, :], Wc2.T, bc2[None, :])
    return logits, emb, probs
